# Initial kernel scaffold; baseline (speedup 1.0000x reference)
import functools

import jax
import jax.numpy as jnp
from jax import lax
from jax.experimental import pallas as pl
from jax.experimental.pallas import tpu as pltpu
from jax.experimental.pallas import tpu_sc as plsc

EMBED = 64
NUM_CORES = 2
NUM_SUBCORES = 16
NW = NUM_CORES * NUM_SUBCORES
CHUNK = 128


@functools.cache
def _make_gather2(nchunk):
    mesh = plsc.VectorSubcoreMesh(core_axis_name="c", subcore_axis_name="s")
    out_t = jax.ShapeDtypeStruct((NW, nchunk, CHUNK, EMBED), jnp.float32)

    @functools.partial(
        pl.kernel,
        mesh=mesh,
        out_type=(out_t, out_t),
        scratch_types=[
            pltpu.VMEM((nchunk, CHUNK), jnp.int32),
            pltpu.VMEM((2, CHUNK, EMBED), jnp.float32),
            pltpu.SemaphoreType.DMA,
            pltpu.SemaphoreType.DMA,
        ],
    )
    def gather2(num_idx, unit_idx, num_tab, unit_tab, out_num, out_unit,
                idx_v, rows_v, sem0, sem1):
        wid = lax.axis_index("s") * NUM_CORES + lax.axis_index("c")
        sems = (sem0, sem1)

        def run_table(idx_hbm, tab, out):
            pltpu.sync_copy(idx_hbm.at[wid], idx_v)
            for b in range(2):
                pltpu.async_copy(tab.at[idx_v.at[b]], rows_v.at[b], sems[b])

            def step(i, carry):
                j = i * 2
                for b in range(2):
                    c = j + b
                    pltpu.make_async_copy(
                        tab.at[idx_v.at[c]], rows_v.at[b], sems[b]).wait()
                    pltpu.sync_copy(rows_v.at[b], out.at[wid, c])

                    @pl.when(c + 2 < nchunk)
                    def _():
                        pltpu.async_copy(
                            tab.at[idx_v.at[c + 2]], rows_v.at[b], sems[b])
                return carry

            lax.fori_loop(0, nchunk // 2, step, 0)

        run_table(num_idx, num_tab, out_num)
        run_table(unit_idx, unit_tab, out_unit)

    return gather2


def kernel(num_tokens, unit_tokens, num_table, unit_table):
    B, S = num_tokens.shape
    rows = B * S
    assert rows % (NW * CHUNK) == 0
    nchunk = rows // (NW * CHUNK)
    ni = num_tokens.reshape(NW, nchunk, CHUNK).astype(jnp.int32)
    ui = unit_tokens.reshape(NW, nchunk, CHUNK).astype(jnp.int32)
    out_num, out_unit = _make_gather2(nchunk)(ni, ui, num_table, unit_table)
    return (out_num.reshape(B, S, EMBED), out_unit.reshape(B, S, EMBED))


# speedup vs baseline: 4.8479x; 4.8479x over previous
import functools

import jax
import jax.numpy as jnp
from jax import lax
from jax.experimental import pallas as pl
from jax.experimental.pallas import tpu as pltpu
from jax.experimental.pallas import tpu_sc as plsc

EMBED = 64
NUM_CORES = 2
NUM_SUBCORES = 16
NW = NUM_CORES * NUM_SUBCORES
CHUNK = 128


@functools.cache
def _make_gather2(nchunk):
    mesh = plsc.VectorSubcoreMesh(core_axis_name="c", subcore_axis_name="s")
    out_t = jax.ShapeDtypeStruct((NW, nchunk, CHUNK, EMBED), jnp.float32)

    @functools.partial(
        pl.kernel,
        mesh=mesh,
        out_type=(out_t, out_t),
        compiler_params=pltpu.CompilerParams(use_tc_tiling_on_sc=False),
        scratch_types=[
            pltpu.VMEM((nchunk, CHUNK), jnp.int32),
            pltpu.VMEM((2, CHUNK, EMBED), jnp.float32),
            pltpu.SemaphoreType.DMA,
            pltpu.SemaphoreType.DMA,
        ],
    )
    def gather2(num_idx, unit_idx, num_tab, unit_tab, out_num, out_unit,
                idx_v, rows_v, sem0, sem1):
        wid = lax.axis_index("s") * NUM_CORES + lax.axis_index("c")
        sems = (sem0, sem1)

        def run_table(idx_hbm, tab, out):
            pltpu.sync_copy(idx_hbm.at[wid], idx_v)
            for b in range(2):
                pltpu.async_copy(tab.at[idx_v.at[b]], rows_v.at[b], sems[b])

            def step(i, carry):
                j = i * 2
                for b in range(2):
                    c = j + b
                    pltpu.make_async_copy(
                        tab.at[idx_v.at[c]], rows_v.at[b], sems[b]).wait()
                    pltpu.sync_copy(rows_v.at[b], out.at[wid, c])

                    @pl.when(c + 2 < nchunk)
                    def _():
                        pltpu.async_copy(
                            tab.at[idx_v.at[c + 2]], rows_v.at[b], sems[b])
                return carry

            lax.fori_loop(0, nchunk // 2, step, 0)

        run_table(num_idx, num_tab, out_num)
        run_table(unit_idx, unit_tab, out_unit)

    return gather2


def kernel(num_tokens, unit_tokens, num_table, unit_table):
    B, S = num_tokens.shape
    rows = B * S
    assert rows % (NW * CHUNK) == 0
    nchunk = rows // (NW * CHUNK)
    ni = num_tokens.reshape(NW, nchunk, CHUNK).astype(jnp.int32)
    ui = unit_tokens.reshape(NW, nchunk, CHUNK).astype(jnp.int32)
    out_num, out_unit = _make_gather2(nchunk)(ni, ui, num_table, unit_table)
    return (out_num.reshape(B, S, EMBED), out_unit.reshape(B, S, EMBED))


# speedup vs baseline: 4.9972x; 1.0308x over previous
import functools

import jax
import jax.numpy as jnp
from jax import lax
from jax.experimental import pallas as pl
from jax.experimental.pallas import tpu as pltpu
from jax.experimental.pallas import tpu_sc as plsc

EMBED = 64
NUM_CORES = 2
NUM_SUBCORES = 16
NW = NUM_CORES * NUM_SUBCORES
CHUNK = 128
NBUF = 5


@functools.cache
def _make_gather2(nchunk):
    assert nchunk % NBUF == 0 and nchunk > NBUF
    mesh = plsc.VectorSubcoreMesh(core_axis_name="c", subcore_axis_name="s")
    out_t = jax.ShapeDtypeStruct((NW, nchunk, CHUNK, EMBED), jnp.float32)

    @functools.partial(
        pl.kernel,
        mesh=mesh,
        out_type=(out_t, out_t),
        compiler_params=pltpu.CompilerParams(use_tc_tiling_on_sc=False),
        scratch_types=[
            pltpu.VMEM((nchunk, CHUNK), jnp.int32),
            pltpu.VMEM((NBUF, CHUNK, EMBED), jnp.float32),
        ]
        + [pltpu.SemaphoreType.DMA] * (2 * NBUF),
    )
    def gather2(num_idx, unit_idx, num_tab, unit_tab, out_num, out_unit,
                idx_v, rows_v, *sems):
        wid = lax.axis_index("s") * NUM_CORES + lax.axis_index("c")
        sem_g = sems[:NBUF]
        sem_w = sems[NBUF:]

        def fire(tab, b, c):
            pltpu.async_copy(tab.at[idx_v.at[c]], rows_v.at[b], sem_g[b])

        def drain(tab, b, c):
            pltpu.make_async_copy(
                tab.at[idx_v.at[c]], rows_v.at[b], sem_g[b]).wait()

        def put(out, b, c):
            pltpu.async_copy(rows_v.at[b], out.at[wid, c], sem_w[b])

        def put_wait(out, b, c):
            pltpu.make_async_copy(rows_v.at[b], out.at[wid, c], sem_w[b]).wait()

        def run_table(idx_hbm, tab, out):
            pltpu.sync_copy(idx_hbm.at[wid], idx_v)
            for c in range(NBUF - 1):
                fire(tab, c, c)

            def step(i, carry):
                for b in range(NBUF):
                    c = i * NBUF + b
                    drain(tab, b, c)
                    put(out, b, c)
                    bf = (b + NBUF - 1) % NBUF

                    @pl.when(c + NBUF - 1 < nchunk)
                    def _():
                        @pl.when(c >= 1)
                        def _():
                            put_wait(out, bf, c - 1)

                        fire(tab, bf, c + NBUF - 1)
                return carry

            lax.fori_loop(0, nchunk // NBUF, step, 0)
            for b in range(NBUF):
                put_wait(out, b, nchunk - NBUF + b)

        run_table(num_idx, num_tab, out_num)
        run_table(unit_idx, unit_tab, out_unit)

    return gather2


def kernel(num_tokens, unit_tokens, num_table, unit_table):
    B, S = num_tokens.shape
    rows = B * S
    assert rows % (NW * CHUNK) == 0
    nchunk = rows // (NW * CHUNK)
    ni = num_tokens.reshape(NW, nchunk, CHUNK).astype(jnp.int32)
    ui = unit_tokens.reshape(NW, nchunk, CHUNK).astype(jnp.int32)
    out_num, out_unit = _make_gather2(nchunk)(ni, ui, num_table, unit_table)
    return (out_num.reshape(B, S, EMBED), out_unit.reshape(B, S, EMBED))
